# Initial kernel scaffold; baseline (speedup 1.0000x reference)
import jax
import jax.numpy as jnp
from jax.experimental import pallas as pl


def kernel(x, edge_index, batch, W1_rel, W1_root, b1, p1, W2_rel, W2_root, b2, p2, W_l1, b_l1, W_l2, b_l2, W_l3, b_l3):
    raise NotImplementedError("write your pallas kernel here")


# speedup vs baseline: 4.6323x; 4.6323x over previous
import functools

import jax
import jax.numpy as jnp
from jax import lax
from jax.experimental import pallas as pl
from jax.experimental.pallas import tpu as pltpu
from jax.experimental.pallas import tpu_sc as plsc

_N = 50000
_E = 800000
_B = 50
_NPG = 1000
_K1 = 800
_K2 = 640
_NHID = 128
_NSC = 2
_NTILES = 16
_CHUNK = 128
_NPAD = 50048
_RPT = _NPAD // _NTILES
_ESC = _E // _NSC
_NCH = _ESC // _CHUNK


def _sc_mesh():
    return plsc.VectorSubcoreMesh(core_axis_name="c", subcore_axis_name="s")


def _zero_vmem(ref, nrows, ncol16):
    z = jnp.zeros((16,), jnp.float32)

    def body(i, carry):
        for j in range(ncol16):
            ref[i, pl.ds(j * 16, 16)] = z
        return carry

    lax.fori_loop(0, nrows, body, 0)


def _seg_accumulate(tab_hbm, src_hbm, dst_hbm, out_hbm, sc, tile,
                    src_v, dst_v, rows_v, zbuf_v, acc_sh, sem):
    r0 = tile * _RPT
    pltpu.sync_copy(zbuf_v, acc_sh.at[pl.ds(r0, _RPT), :])
    plsc.subcore_barrier()
    nmine = (_NCH - tile + _NTILES - 1) // _NTILES
    ebase = sc * _ESC

    def body(ki, carry):
        off = ebase + (tile + ki * _NTILES) * _CHUNK
        pltpu.sync_copy(src_hbm.at[pl.ds(off, _CHUNK)], src_v)
        pltpu.sync_copy(dst_hbm.at[pl.ds(off, _CHUNK)], dst_v)
        pltpu.async_copy(tab_hbm.at[src_v], rows_v, sem).wait()
        pltpu.sync_copy(rows_v, acc_sh.at[dst_v], add=True)
        return carry

    lax.fori_loop(0, nmine, body, 0)
    plsc.subcore_barrier()
    pltpu.sync_copy(acc_sh.at[pl.ds(r0, _RPT), :],
                    out_hbm.at[sc, pl.ds(r0, _RPT), :])
    plsc.subcore_barrier()


def _seg16_body(x16, src, dst, out, src_v, dst_v, rows_v, zbuf_v, acc_sh, sem):
    sc = lax.axis_index("c")
    tile = lax.axis_index("s")
    _zero_vmem(zbuf_v, _RPT, 1)
    _seg_accumulate(x16, src, dst, out, sc, tile,
                    src_v, dst_v, rows_v, zbuf_v, acc_sh, sem)


def _seg16(x16, src, dst):
    return pl.kernel(
        _seg16_body,
        out_type=jax.ShapeDtypeStruct((_NSC, _NPAD, 16), jnp.float32),
        mesh=_sc_mesh(),
        compiler_params=pltpu.CompilerParams(use_tc_tiling_on_sc=False),
        scratch_types=[
            pltpu.VMEM((_CHUNK,), jnp.int32),
            pltpu.VMEM((_CHUNK,), jnp.int32),
            pltpu.VMEM((_CHUNK, 16), jnp.float32),
            pltpu.VMEM((_RPT, 16), jnp.float32),
            pltpu.VMEM_SHARED((_NPAD, 16), jnp.float32),
            pltpu.SemaphoreType.DMA,
        ],
    )(x16, src, dst)


def _seg32_body(c0, c1, c2, c3, c4, c5, c6, c7, src, dst,
                o0, o1, o2, o3, o4, o5, o6, o7,
                src_v, dst_v, rows_v, zbuf_v, acc_sh, sem):
    sc = lax.axis_index("c")
    tile = lax.axis_index("s")
    _zero_vmem(zbuf_v, _RPT, 1)
    for tab, out in ((c0, o0), (c1, o1), (c2, o2), (c3, o3),
                     (c4, o4), (c5, o5), (c6, o6), (c7, o7)):
        _seg_accumulate(tab, src, dst, out, sc, tile,
                        src_v, dst_v, rows_v, zbuf_v, acc_sh, sem)


def _seg32(cs, src, dst):
    shp = jax.ShapeDtypeStruct((_NSC, _NPAD, 16), jnp.float32)
    return pl.kernel(
        _seg32_body,
        out_type=(shp,) * 8,
        mesh=_sc_mesh(),
        compiler_params=pltpu.CompilerParams(use_tc_tiling_on_sc=False),
        scratch_types=[
            pltpu.VMEM((_CHUNK,), jnp.int32),
            pltpu.VMEM((_CHUNK,), jnp.int32),
            pltpu.VMEM((_CHUNK, 16), jnp.float32),
            pltpu.VMEM((_RPT, 16), jnp.float32),
            pltpu.VMEM_SHARED((_NPAD, 16), jnp.float32),
            pltpu.SemaphoreType.DMA,
        ],
    )(*cs, src, dst)


def _t1_body(aggp_ref, x_ref, wr_ref, wo_ref, b_ref, p_ref, h_ref, s_ref):
    agg = aggp_ref[0] + aggp_ref[1]
    h = jnp.dot(agg, wr_ref[...], preferred_element_type=jnp.float32)
    h += jnp.dot(x_ref[...], wo_ref[...], preferred_element_type=jnp.float32)
    h = jnp.maximum(h + b_ref[...], 0.0)
    h_ref[...] = h
    p = p_ref[...]
    rnorm = lax.rsqrt(jnp.sum(p * p))
    s_ref[...] = jnp.dot(h, p, preferred_element_type=jnp.float32) * rnorm


def _t1(aggp, x16, wr, wo, b, p):
    nb = 1000
    return pl.pallas_call(
        _t1_body,
        grid=(_N // nb,),
        in_specs=[
            pl.BlockSpec((_NSC, nb, 16), lambda i: (0, i, 0)),
            pl.BlockSpec((nb, 16), lambda i: (i, 0)),
            pl.BlockSpec((16, _NHID), lambda i: (0, 0)),
            pl.BlockSpec((16, _NHID), lambda i: (0, 0)),
            pl.BlockSpec((1, _NHID), lambda i: (0, 0)),
            pl.BlockSpec((_NHID, 1), lambda i: (0, 0)),
        ],
        out_specs=[
            pl.BlockSpec((nb, _NHID), lambda i: (i, 0)),
            pl.BlockSpec((nb, 1), lambda i: (i, 0)),
        ],
        out_shape=[
            jax.ShapeDtypeStruct((_N, _NHID), jnp.float32),
            jax.ShapeDtypeStruct((_N, 1), jnp.float32),
        ],
    )(aggp, x16, wr, wo, b, p)


def _topk_body(K, s_ref, tsc_ref, m_ref):
    s = s_ref[...]
    b = lax.bitcast_convert_type(s, jnp.uint32)
    u = jnp.where(b >= jnp.uint32(0x80000000), ~b, b | jnp.uint32(0x80000000))
    t = jnp.zeros((_B, 1), jnp.uint32)
    for bit in range(31, -1, -1):
        cand = t | jnp.uint32(1 << bit)
        cnt = jnp.sum((u >= cand).astype(jnp.int32), axis=1, keepdims=True)
        t = jnp.where(cnt >= K, cand, t)
    gt = u > t
    cnt_gt = jnp.sum(gt.astype(jnp.int32), axis=1, keepdims=True)
    need = K - cnt_gt
    eq = u == t
    iota = lax.broadcasted_iota(jnp.int32, (_B, _NPG), 1)
    lo = jnp.zeros((_B, 1), jnp.int32)
    hi = jnp.full((_B, 1), _NPG, jnp.int32)
    for _ in range(11):
        mid = (lo + hi) // 2
        ce = jnp.sum((eq & (iota < mid)).astype(jnp.int32), axis=1,
                     keepdims=True)
        ok = ce >= need
        hi = jnp.where(ok, mid, hi)
        lo = jnp.where(ok, lo, mid + 1)
    mask = gt | (eq & (iota < hi))
    tsc_ref[...] = jnp.where(mask, jnp.tanh(s), 0.0)
    m_ref[...] = mask.astype(jnp.float32)


def _topk(s2d, K):
    return pl.pallas_call(
        functools.partial(_topk_body, K),
        out_shape=[
            jax.ShapeDtypeStruct((_B, _NPG), jnp.float32),
            jax.ShapeDtypeStruct((_B, _NPG), jnp.float32),
        ],
    )(s2d)


def _t3_body(h_ref, tsc_ref, m_ref, *orefs):
    cb = h_ref[...] * tsc_ref[...]
    for f in range(8):
        orefs[f][...] = cb[:, f * 16:(f + 1) * 16]
    mean = jnp.sum(cb, axis=0, keepdims=True) * (1.0 / _K1)
    neg = jnp.float32(-jnp.inf)
    mx = jnp.max(jnp.where(m_ref[...] > 0.5, cb, neg), axis=0, keepdims=True)
    orefs[8][...] = jnp.concatenate([mean, mx], axis=1)[None]


def _t3(h, tsc_col, m_col):
    nb = 1000
    cshp = jax.ShapeDtypeStruct((_N, 16), jnp.float32)
    cspec = pl.BlockSpec((nb, 16), lambda i: (i, 0))
    return pl.pallas_call(
        _t3_body,
        grid=(_N // nb,),
        in_specs=[
            pl.BlockSpec((nb, _NHID), lambda i: (i, 0)),
            pl.BlockSpec((nb, 1), lambda i: (i, 0)),
            pl.BlockSpec((nb, 1), lambda i: (i, 0)),
        ],
        out_specs=[cspec] * 8 + [pl.BlockSpec((1, 1, 2 * _NHID),
                                              lambda i: (i, 0, 0))],
        out_shape=[cshp] * 8 + [jax.ShapeDtypeStruct((_B, 1, 2 * _NHID),
                                                     jnp.float32)],
    )(h, tsc_col, m_col)


def _t4_body(*refs):
    accs = refs[0:8]
    cs = refs[8:16]
    wr_ref, wo_ref, b_ref, p_ref, m_ref, g_ref, s_ref = refs[16:]
    agg = jnp.concatenate([a[0] + a[1] for a in accs], axis=1)
    cc = jnp.concatenate([c[...] for c in cs], axis=1)
    g = jnp.dot(agg, wr_ref[...], preferred_element_type=jnp.float32)
    g += jnp.dot(cc, wo_ref[...], preferred_element_type=jnp.float32)
    g = jnp.maximum(g + b_ref[...], 0.0)
    g_ref[...] = g
    p = p_ref[...]
    rnorm = lax.rsqrt(jnp.sum(p * p))
    s2 = jnp.dot(g, p, preferred_element_type=jnp.float32) * rnorm
    s_ref[...] = jnp.where(m_ref[...] > 0.5, s2, jnp.float32(-jnp.inf))


def _t4(accs, cs, wr, wo, b, p, m_col):
    nb = 1000
    aspec = pl.BlockSpec((_NSC, nb, 16), lambda i: (0, i, 0))
    cspec = pl.BlockSpec((nb, 16), lambda i: (i, 0))
    return pl.pallas_call(
        _t4_body,
        grid=(_N // nb,),
        in_specs=[aspec] * 8 + [cspec] * 8 + [
                  pl.BlockSpec((_NHID, _NHID), lambda i: (0, 0)),
                  pl.BlockSpec((_NHID, _NHID), lambda i: (0, 0)),
                  pl.BlockSpec((1, _NHID), lambda i: (0, 0)),
                  pl.BlockSpec((_NHID, 1), lambda i: (0, 0)),
                  pl.BlockSpec((nb, 1), lambda i: (i, 0))],
        out_specs=[pl.BlockSpec((nb, _NHID), lambda i: (i, 0)),
                   pl.BlockSpec((nb, 1), lambda i: (i, 0))],
        out_shape=[jax.ShapeDtypeStruct((_N, _NHID), jnp.float32),
                   jax.ShapeDtypeStruct((_N, 1), jnp.float32)],
    )(*accs, *cs, wr, wo, b, p, m_col)


def _t6_body(g_ref, tsc_ref, m_ref, x2_ref):
    gb = g_ref[...] * tsc_ref[...]
    mean = jnp.sum(gb, axis=0, keepdims=True) * (1.0 / _K2)
    neg = jnp.float32(-jnp.inf)
    mx = jnp.max(jnp.where(m_ref[...] > 0.5, gb, neg), axis=0, keepdims=True)
    x2_ref[...] = jnp.concatenate([mean, mx], axis=1)[None]


def _t6(g, tsc_col, m_col):
    nb = 1000
    return pl.pallas_call(
        _t6_body,
        grid=(_N // nb,),
        in_specs=[
            pl.BlockSpec((nb, _NHID), lambda i: (i, 0)),
            pl.BlockSpec((nb, 1), lambda i: (i, 0)),
            pl.BlockSpec((nb, 1), lambda i: (i, 0)),
        ],
        out_specs=pl.BlockSpec((1, 1, 2 * _NHID), lambda i: (i, 0, 0)),
        out_shape=jax.ShapeDtypeStruct((_B, 1, 2 * _NHID), jnp.float32),
    )(g, tsc_col, m_col)


def _t7_body(x1_ref, x2_ref, w1_ref, b1_ref, w2_ref, b2_ref, w3_ref, b3_ref,
             o_ref):
    z = x1_ref[...] + x2_ref[...]
    z = jnp.maximum(
        jnp.dot(z, w1_ref[...], preferred_element_type=jnp.float32)
        + b1_ref[...], 0.0)
    z = jnp.maximum(
        jnp.dot(z, w2_ref[...], preferred_element_type=jnp.float32)
        + b2_ref[...], 0.0)
    z = jnp.dot(z, w3_ref[...], preferred_element_type=jnp.float32) + b3_ref[...]
    m = jnp.max(z, axis=1, keepdims=True)
    lse = m + jnp.log(jnp.sum(jnp.exp(z - m), axis=1, keepdims=True))
    o_ref[...] = z - lse


def _t7(x1, x2, w1, b1, w2, b2, w3, b3):
    return pl.pallas_call(
        _t7_body,
        out_shape=jax.ShapeDtypeStruct((_B, 2), jnp.float32),
    )(x1, x2, w1, b1, w2, b2, w3, b3)


def kernel(x, edge_index, batch, W1_rel, W1_root, b1, p1, W2_rel, W2_root,
           b2, p2, W_l1, b_l1, W_l2, b_l2, W_l3, b_l3):
    src = edge_index[0]
    dst = edge_index[1]
    x16 = jnp.pad(x, ((0, 0), (0, 2)))
    w1r = jnp.pad(W1_rel, ((0, 2), (0, 0)))
    w1o = jnp.pad(W1_root, ((0, 2), (0, 0)))

    aggp = _seg16(x16, src, dst)
    h, s_col = _t1(aggp, x16, w1r, w1o, b1.reshape(1, -1), p1.reshape(-1, 1))
    tsc1, m1 = _topk(s_col.reshape(_B, _NPG), _K1)
    tsc1c = tsc1.reshape(_N, 1)
    m1c = m1.reshape(_N, 1)
    *cs, x1 = _t3(h, tsc1c, m1c)
    accs = _seg32(cs, src, dst)
    g, s2_col = _t4(accs, cs, W2_rel, W2_root,
                    b2.reshape(1, -1), p2.reshape(-1, 1), m1c)
    tsc2, m2 = _topk(s2_col.reshape(_B, _NPG), _K2)
    x2 = _t6(g, tsc2.reshape(_N, 1), m2.reshape(_N, 1))
    return _t7(x1.reshape(_B, -1), x2.reshape(_B, -1),
               W_l1, b_l1.reshape(1, -1), W_l2, b_l2.reshape(1, -1),
               W_l3, b_l3.reshape(1, -1))
